# baseline (speedup 1.0000x reference)
import functools

import jax
import jax.numpy as jnp
from jax.experimental import pallas as pl
from jax.experimental.pallas import tpu as pltpu

_THETA = 10.0
_MAX_ITER = 30
_XT = 3072
_NC = 512
_MC = 256
_LO_SCALE = 512.0
_W_SCALE = 1.0 / 32.0


def _sigmoid(x):
    return 1.0 / (1.0 + jnp.exp(-x))


def _logaddexp0(x):
    return jnp.maximum(x, 0.0) + jnp.log1p(jnp.exp(-jnp.abs(x)))


def _split2(v):
    v_hi = v.astype(jnp.bfloat16)
    v_lo = (v - v_hi.astype(jnp.float32)).astype(jnp.bfloat16)
    return jnp.concatenate([v_hi, v_lo], axis=0)


def _solver_kernel(ah_ref, al_ref, b_ref, c_ref, u_ref, x_ref, eta_ref,
                   at_ref):
    b = b_ref[...]
    c = c_ref[...]
    u = u_ref[...]
    theta_u = _THETA * u
    dtype_eps = float(jnp.finfo(jnp.float32).eps)
    btb = jnp.sum(b * b, axis=-1, keepdims=True)

    m_sz, n_sz = ah_ref.shape

    for j in range(_XT // _NC):
        blk = ah_ref[:, j * _NC:(j + 1) * _NC]
        at_ref[j * _NC:(j + 1) * _NC, :] = jnp.swapaxes(blk, 0, 1)

    def At_mul(w):
        w2 = _split2(w)
        w8 = (w * _W_SCALE).astype(jnp.float8_e4m3fn)
        acc = None
        for i in range(m_sz // _MC):
            sl = slice(i * _MC, (i + 1) * _MC)
            hi = jax.lax.dot_general(
                w2[:, sl], ah_ref[sl, :],
                (((1,), (0,)), ((), ())), preferred_element_type=jnp.float32)
            lo = jax.lax.dot_general(
                w8[:, sl], al_ref[sl, :],
                (((1,), (0,)), ((), ())), preferred_element_type=jnp.float32)
            part = (hi[0:1, :] + hi[1:2, :]
                    + lo * (1.0 / (_LO_SCALE * _W_SCALE)))
            acc = part if acc is None else acc + part
        return acc

    def A_mul(v):
        v2 = _split2(v)
        v8 = v.astype(jnp.float8_e4m3fn)
        acc = None
        for j in range(_XT // _NC):
            sl = slice(j * _NC, (j + 1) * _NC)
            part = jax.lax.dot_general(
                v2[:, sl], at_ref[sl, :],
                (((1,), (0,)), ((), ())), preferred_element_type=jnp.float32)
            part = part[0:1, :] + part[1:2, :]
            acc = part if acc is None else acc + part
        for j in range(_XT // _NC, n_sz // _NC):
            sl = slice(j * _NC, (j + 1) * _NC)
            part = jax.lax.dot_general(
                v2[:, sl], ah_ref[:, sl],
                (((1,), (1,)), ((), ())), preferred_element_type=jnp.float32)
            acc = acc + part[0:1, :] + part[1:2, :]
        lo_acc = None
        for j in range(n_sz // _NC):
            sl = slice(j * _NC, (j + 1) * _NC)
            lo = jax.lax.dot_general(
                v8[:, sl], al_ref[:, sl],
                (((1,), (1,)), ((), ())), preferred_element_type=jnp.float32)
            lo_acc = lo if lo_acc is None else lo_acc + lo
        return acc + lo_acc * (1.0 / _LO_SCALE)

    M = jnp.full((1, 1), _THETA, dtype=jnp.float32)
    beta_old = jnp.zeros((1, 1), dtype=jnp.float32)
    last_cond = jnp.zeros((1, 1), dtype=jnp.float32)
    eta = jnp.zeros((1, m_sz), dtype=jnp.float32)
    zeta = jnp.zeros((1, m_sz), dtype=jnp.float32)
    p_eta = jnp.zeros((1, n_sz), dtype=jnp.float32)
    p_zeta = jnp.zeros((1, n_sz), dtype=jnp.float32)
    x_final_pu = _sigmoid(-c * theta_u)

    def body(_, carry):
        (M, beta_old, last_cond, eta, zeta, p_eta, p_zeta, x_final_pu) = carry
        alpha = 0.5 / M + jnp.sqrt((0.25 / M + beta_old) / M)
        beta_new = beta_old + alpha
        tau = alpha / beta_new
        p_lam = p_eta + tau * (p_zeta - p_eta)
        neg_l = -(c - p_lam) * theta_u
        x_lam = _sigmoid(neg_l)
        q = A_mul(u * x_lam)
        grad = q - b
        zeta_new = zeta - alpha * grad
        eta_new = eta + tau * (zeta_new - eta)
        t = At_mul(grad)
        p_zeta_new = p_zeta - alpha * t
        p_eta_new = p_eta + tau * (p_zeta_new - p_eta)
        neg_e = -(c - p_eta_new) * theta_u
        gap = (jnp.sum(q * q, axis=-1, keepdims=True) - btb) * (0.5 / M) + (
            jnp.sum(_logaddexp0(neg_e) - _logaddexp0(neg_l),
                    axis=-1, keepdims=True) / _THETA)
        cond = (gap <= dtype_eps).astype(jnp.float32)
        cond_b = cond > 0.5
        M = jnp.maximum(
            jnp.where(cond_b, jnp.where(last_cond > 0.5, M * 0.5, M), M * 2.0),
            dtype_eps)
        beta_old = jnp.where(cond_b, beta_new, beta_old)
        eta = jnp.where(cond_b, eta_new, eta)
        zeta = jnp.where(cond_b, zeta_new, zeta)
        p_eta = jnp.where(cond_b, p_eta_new, p_eta)
        p_zeta = jnp.where(cond_b, p_zeta_new, p_zeta)
        x_final_pu = jnp.where(cond_b, x_final_pu + tau * (x_lam - x_final_pu),
                               x_final_pu)
        return (M, beta_old, cond, eta, zeta, p_eta, p_zeta, x_final_pu)

    carry = (M, beta_old, last_cond, eta, zeta, p_eta, p_zeta, x_final_pu)
    carry = jax.lax.fori_loop(0, _MAX_ITER, body, carry)
    (_, _, _, eta, _, _, _, x_final_pu) = carry
    x_ref[...] = u * x_final_pu
    eta_ref[...] = eta


@jax.jit
def kernel(A, b, c, u):
    m, n = A.shape
    a_hi = A.astype(jnp.bfloat16)
    a_lo = ((A - a_hi.astype(jnp.float32)) * _LO_SCALE).astype(
        jnp.float8_e4m3fn)
    x_final, eta = pl.pallas_call(
        _solver_kernel,
        out_shape=(jax.ShapeDtypeStruct((1, n), jnp.float32),
                   jax.ShapeDtypeStruct((1, m), jnp.float32)),
        scratch_shapes=[pltpu.VMEM((_XT, m), jnp.bfloat16)],
        compiler_params=pltpu.CompilerParams(
            vmem_limit_bytes=100 * 1024 * 1024),
    )(a_hi, a_lo, b, c, u)
    return (x_final, eta)
